# Initial kernel scaffold; baseline (speedup 1.0000x reference)
import jax
import jax.numpy as jnp
from jax.experimental import pallas as pl


def kernel(mem_positions, mem_colors, mem_scales, mem_opacities, T_obj_world, positions, colors, scales, opacities, idx):
    raise NotImplementedError("write your pallas kernel here")


# speedup vs baseline: 8.4942x; 8.4942x over previous
import functools

import jax
import jax.numpy as jnp
from jax import lax
from jax.experimental import pallas as pl
from jax.experimental.pallas import tpu as pltpu
from jax.experimental.pallas import tpu_sc as plsc

M = 300000
B = 100000
NW = 32
L = 16

BW = 3136
SM = 9376
CH = 2000
GG = 112

_mesh = plsc.VectorSubcoreMesh(core_axis_name="c", subcore_axis_name="s")
_cparams = pltpu.CompilerParams(needs_layout_passes=False,
                                use_tc_tiling_on_sc=False)


def _wid():
    return lax.axis_index("c") * 16 + lax.axis_index("s")


@functools.partial(
    pl.kernel,
    out_type=(
        jax.ShapeDtypeStruct((B * 10,), jnp.float32),
        jax.ShapeDtypeStruct((M,), jnp.int32),
    ),
    mesh=_mesh,
    compiler_params=_cparams,
    scratch_types=[
        pltpu.VMEM((BW * 3,), jnp.float32),
        pltpu.VMEM((BW * 3,), jnp.float32),
        pltpu.VMEM((BW * 3,), jnp.float32),
        pltpu.VMEM((BW,), jnp.float32),
        pltpu.VMEM((BW * 10,), jnp.float32),
        pltpu.VMEM((L,), jnp.float32),
        pltpu.VMEM((CH,), jnp.int32),
        pltpu.VMEM((SM,), jnp.int32),
    ],
)
def _k1(posf, colf, sclf, opaf, tmat, idxh, rows_out, s_out,
        pos_v, col_v, scl_v, opa_v, rows_v, t_v, idx_v, s_v):
    wid = _wid()
    base = pl.multiple_of(jnp.minimum(wid * BW, B - BW), 16)
    lanes = lax.iota(jnp.int32, L)
    i3 = lanes * 3
    i10 = lanes * 10

    pltpu.sync_copy(tmat, t_v)
    pltpu.sync_copy(posf.at[pl.ds(base * 3, BW * 3)], pos_v)
    pltpu.sync_copy(colf.at[pl.ds(base * 3, BW * 3)], col_v)
    pltpu.sync_copy(sclf.at[pl.ds(base * 3, BW * 3)], scl_v)
    pltpu.sync_copy(opaf.at[pl.ds(base, BW)], opa_v)

    tv = t_v[...]
    zf = jnp.zeros((L,), jnp.float32)
    t = [jnp.broadcast_to(jnp.sum(jnp.where(lanes == k, tv, zf)), (L,))
         for k in range(12)]

    def grp(g, carry):
        r3 = g * (L * 3)
        r10 = g * (L * 10)
        src = i3 + r3
        px = plsc.load_gather(pos_v, [src])
        py = plsc.load_gather(pos_v, [src + 1])
        pz = plsc.load_gather(pos_v, [src + 2])
        pwx = t[0] * px + t[1] * py + t[2] * pz + t[3]
        pwy = t[4] * px + t[5] * py + t[6] * pz + t[7]
        pwz = t[8] * px + t[9] * py + t[10] * pz + t[11]
        ob = i10 + r10
        plsc.store_scatter(rows_v, [ob], pwx)
        plsc.store_scatter(rows_v, [ob + 1], pwy)
        plsc.store_scatter(rows_v, [ob + 2], pwz)
        for k in range(3):
            plsc.store_scatter(rows_v, [ob + 3 + k], plsc.load_gather(col_v, [src + k]))
        for k in range(3):
            plsc.store_scatter(rows_v, [ob + 6 + k], plsc.load_gather(scl_v, [src + k]))
        op = plsc.load_gather(opa_v, [lanes + g * L])
        plsc.store_scatter(rows_v, [ob + 9], op)
        return carry

    lax.fori_loop(0, BW // L, grp, 0)
    pltpu.sync_copy(rows_v, rows_out.at[pl.ds(base * 10, BW * 10)])

    sbase = pl.multiple_of(jnp.minimum(wid * SM, M - SM), 16)

    def chunk(c, carry):
        pltpu.sync_copy(idxh.at[pl.ds(c * CH, CH)], idx_v)

        def vec(v, carry2):
            iv = plsc.load_gather(idx_v, [lanes + v * L])
            jv = lanes + (c * CH + v * L)
            sl = iv - sbase
            m = (sl >= 0) & (sl < SM)
            slc = jnp.clip(sl, 0, SM - 1)
            plsc.store_scatter(s_v, [slc], jv, mask=m)
            g1 = plsc.load_gather(s_v, [slc], mask=m)
            m2 = m & (g1 < jv)
            plsc.store_scatter(s_v, [slc], jv, mask=m2)
            g2 = plsc.load_gather(s_v, [slc], mask=m2)
            m3 = m2 & (g2 < jv)
            plsc.store_scatter(s_v, [slc], jv, mask=m3)
            return carry2

        lax.fori_loop(0, CH // L, vec, 0)
        return carry

    lax.fori_loop(0, B // CH, chunk, 0)
    pltpu.sync_copy(s_v, s_out.at[pl.ds(sbase, SM)])


@functools.partial(
    pl.kernel,
    out_type=jax.ShapeDtypeStruct((B * 10,), jnp.float32),
    mesh=_mesh,
    compiler_params=_cparams,
    scratch_types=[
        pltpu.VMEM((BW,), jnp.int32),
        pltpu.VMEM((BW,), jnp.int32),
        pltpu.VMEM((BW * 10,), jnp.int32),
        pltpu.VMEM((BW * 10,), jnp.float32),
        pltpu.SemaphoreType.DMA,
    ],
)
def _k2(s_hbm, rowsf_hbm, idxh, out_hbm, iv_v, wv_v, idx10_v, orow_v, sem):
    wid = _wid()
    base = pl.multiple_of(jnp.minimum(wid * BW, B - BW), 16)
    lanes = lax.iota(jnp.int32, L)
    i10 = lanes * 10

    pltpu.sync_copy(idxh.at[pl.ds(base, BW)], iv_v)

    ds = []
    for c in range(BW // GG):
        ds.append(pltpu.async_copy(
            s_hbm.at[iv_v.at[pl.ds(c * GG, GG)]],
            wv_v.at[pl.ds(c * GG, GG)], sem))
    for d in ds:
        d.wait()

    def grp(g, carry):
        wv = plsc.load_gather(wv_v, [lanes + g * L])
        w10 = wv * 10
        ob = i10 + g * (L * 10)
        for c in range(10):
            plsc.store_scatter(idx10_v, [ob + c], w10 + c)
        return carry

    lax.fori_loop(0, BW // L, grp, 0)

    ds = []
    for c in range(BW * 10 // GG):
        ds.append(pltpu.async_copy(
            rowsf_hbm.at[idx10_v.at[pl.ds(c * GG, GG)]],
            orow_v.at[pl.ds(c * GG, GG)], sem))
    for d in ds:
        d.wait()

    pltpu.sync_copy(orow_v, out_hbm.at[pl.ds(base * 10, BW * 10)])


def kernel(mem_positions, mem_colors, mem_scales, mem_opacities, T_obj_world,
           positions, colors, scales, opacities, idx):
    idx32 = idx.astype(jnp.int32)
    rows_flat, s = _k1(
        positions.reshape(-1), colors.reshape(-1), scales.reshape(-1),
        opacities.reshape(-1), T_obj_world.reshape(-1), idx32)
    return _k2(s, rows_flat, idx32).reshape(B, 10)


# speedup vs baseline: 12.9098x; 1.5198x over previous
import functools

import jax
import jax.numpy as jnp
from jax import lax
from jax.experimental import pallas as pl
from jax.experimental.pallas import tpu as pltpu
from jax.experimental.pallas import tpu_sc as plsc

M = 300000
B = 100000
NW = 32
L = 16

BW = 3136
SM = 9376
CH = 2000
GG = 112

_mesh = plsc.VectorSubcoreMesh(core_axis_name="c", subcore_axis_name="s")
_cparams = pltpu.CompilerParams(needs_layout_passes=False,
                                use_tc_tiling_on_sc=False)


def _wid():
    return lax.axis_index("c") * 16 + lax.axis_index("s")


@functools.partial(
    pl.kernel,
    out_type=jax.ShapeDtypeStruct((M,), jnp.int32),
    mesh=_mesh,
    compiler_params=_cparams,
    scratch_types=[
        pltpu.VMEM((B,), jnp.int32),
        pltpu.VMEM((SM,), jnp.int32),
    ],
)
def _k_scan(idxh, s_out, idx_v, s_v):
    wid = _wid()
    lanes = lax.iota(jnp.int32, L)
    sbase = pl.multiple_of(jnp.minimum(wid * SM, M - SM), 16)
    pltpu.sync_copy(idxh, idx_v)

    def chunk(c, carry):
        cb = pl.multiple_of(c * CH, 16)
        for v in range(CH // L):
            iv = idx_v[pl.ds(cb + v * L, L)]
            jv = lanes + (cb + v * L)
            sl = iv - sbase
            m = plsc.bitcast(sl, jnp.uint32) < jnp.uint32(SM)
            slc = jnp.where(m, sl, 0)
            plsc.store_scatter(s_v, [slc], jv, mask=m)
            g1 = plsc.load_gather(s_v, [slc], mask=m)
            m2 = m & (g1 < jv)
            plsc.store_scatter(s_v, [slc], jv, mask=m2)
        return carry

    lax.fori_loop(0, B // CH, chunk, 0)
    pltpu.sync_copy(s_v, s_out.at[pl.ds(sbase, SM)])


@functools.partial(
    pl.kernel,
    out_type=jax.ShapeDtypeStruct((B * 10,), jnp.float32),
    mesh=_mesh,
    compiler_params=_cparams,
    scratch_types=[
        pltpu.VMEM((BW * 3,), jnp.float32),
        pltpu.VMEM((BW * 3,), jnp.float32),
        pltpu.VMEM((BW * 3,), jnp.float32),
        pltpu.VMEM((BW,), jnp.float32),
        pltpu.VMEM((BW * 10,), jnp.float32),
        pltpu.VMEM((L,), jnp.float32),
    ],
)
def _k_rows(posf, colf, sclf, opaf, tmat, rows_out,
            pos_v, col_v, scl_v, opa_v, rows_v, t_v):
    wid = _wid()
    base = pl.multiple_of(jnp.minimum(wid * BW, B - BW), 16)
    lanes = lax.iota(jnp.int32, L)
    i3 = lanes * 3
    i10 = lanes * 10

    pltpu.sync_copy(tmat, t_v)
    pltpu.sync_copy(posf.at[pl.ds(base * 3, BW * 3)], pos_v)
    pltpu.sync_copy(colf.at[pl.ds(base * 3, BW * 3)], col_v)
    pltpu.sync_copy(sclf.at[pl.ds(base * 3, BW * 3)], scl_v)
    pltpu.sync_copy(opaf.at[pl.ds(base, BW)], opa_v)

    tv = t_v[...]
    zf = jnp.zeros((L,), jnp.float32)
    t = [jnp.broadcast_to(jnp.sum(jnp.where(lanes == k, tv, zf)), (L,))
         for k in range(12)]

    def grp(g, carry):
        r3 = g * (L * 3)
        r10 = g * (L * 10)
        src = i3 + r3
        px = plsc.load_gather(pos_v, [src])
        py = plsc.load_gather(pos_v, [src + 1])
        pz = plsc.load_gather(pos_v, [src + 2])
        pwx = t[0] * px + t[1] * py + t[2] * pz + t[3]
        pwy = t[4] * px + t[5] * py + t[6] * pz + t[7]
        pwz = t[8] * px + t[9] * py + t[10] * pz + t[11]
        ob = i10 + r10
        plsc.store_scatter(rows_v, [ob], pwx)
        plsc.store_scatter(rows_v, [ob + 1], pwy)
        plsc.store_scatter(rows_v, [ob + 2], pwz)
        for k in range(3):
            plsc.store_scatter(rows_v, [ob + 3 + k], plsc.load_gather(col_v, [src + k]))
        for k in range(3):
            plsc.store_scatter(rows_v, [ob + 6 + k], plsc.load_gather(scl_v, [src + k]))
        op = plsc.load_gather(opa_v, [lanes + g * L])
        plsc.store_scatter(rows_v, [ob + 9], op)
        return carry

    lax.fori_loop(0, BW // L, grp, 0)
    pltpu.sync_copy(rows_v, rows_out.at[pl.ds(base * 10, BW * 10)])


@functools.partial(
    pl.kernel,
    out_type=jax.ShapeDtypeStruct((B * 10,), jnp.float32),
    mesh=_mesh,
    compiler_params=_cparams,
    scratch_types=[
        pltpu.VMEM((BW,), jnp.int32),
        pltpu.VMEM((BW,), jnp.int32),
        pltpu.VMEM((BW * 10,), jnp.int32),
        pltpu.VMEM((BW * 10,), jnp.float32),
        pltpu.SemaphoreType.DMA,
    ],
)
def _k2(s_hbm, rowsf_hbm, idxh, out_hbm, iv_v, wv_v, idx10_v, orow_v, sem):
    wid = _wid()
    base = pl.multiple_of(jnp.minimum(wid * BW, B - BW), 16)
    lanes = lax.iota(jnp.int32, L)
    i10 = lanes * 10

    pltpu.sync_copy(idxh.at[pl.ds(base, BW)], iv_v)

    ds = []
    for c in range(BW // GG):
        ds.append(pltpu.async_copy(
            s_hbm.at[iv_v.at[pl.ds(c * GG, GG)]],
            wv_v.at[pl.ds(c * GG, GG)], sem))
    for d in ds:
        d.wait()

    def grp(g, carry):
        wv = plsc.load_gather(wv_v, [lanes + g * L])
        w10 = wv * 10
        ob = i10 + g * (L * 10)
        for c in range(10):
            plsc.store_scatter(idx10_v, [ob + c], w10 + c)
        return carry

    lax.fori_loop(0, BW // L, grp, 0)

    ds = []
    for c in range(BW * 10 // GG):
        ds.append(pltpu.async_copy(
            rowsf_hbm.at[idx10_v.at[pl.ds(c * GG, GG)]],
            orow_v.at[pl.ds(c * GG, GG)], sem))
    for d in ds:
        d.wait()

    pltpu.sync_copy(orow_v, out_hbm.at[pl.ds(base * 10, BW * 10)])


def kernel(mem_positions, mem_colors, mem_scales, mem_opacities, T_obj_world,
           positions, colors, scales, opacities, idx):
    idx32 = idx.astype(jnp.int32)
    s = _k_scan(idx32)
    rows_flat = _k_rows(
        positions.reshape(-1), colors.reshape(-1), scales.reshape(-1),
        opacities.reshape(-1), T_obj_world.reshape(-1))
    return _k2(s, rows_flat, idx32).reshape(B, 10)
